# Initial kernel scaffold; baseline (speedup 1.0000x reference)
import jax
import jax.numpy as jnp
from jax.experimental import pallas as pl


def kernel(nodes, nlist, edges, inv_degree, wq, wk, wv):
    raise NotImplementedError("write your pallas kernel here")


# speedup vs baseline: 1.0935x; 1.0935x over previous
import functools

import jax
import jax.numpy as jnp
from jax import lax
from jax.experimental import pallas as pl
from jax.experimental.pallas import tpu as pltpu
from jax.experimental.pallas import tpu_sc as plsc

N = 10000
K = 16
D_NODE = 256
D_EDGE = 16

L = 16
NC = 2
NS = 16
NW = NC * NS
G = 4
ROWS = G * K
N_PAD = 10240
PER_W = N_PAD // NW
NGRP = PER_W // G
BN = 1000


def _attn_body(nodes_ref, edges_ref, inv_ref, wq_ref, wk_ref, b_ref):
    wqk = jnp.dot(wq_ref[...], wk_ref[...].T, preferred_element_type=jnp.float32)
    r = jnp.dot(nodes_ref[...], wqk, preferred_element_type=jnp.float32)
    r = r * inv_ref[...]
    e = edges_ref[...]
    parts = [
        jnp.sum(e[:, L * j:L * (j + 1)] * r, axis=1, keepdims=True)
        for j in range(K)
    ]
    qd = jnp.concatenate(parts, axis=1)
    m = jnp.max(qd, axis=1, keepdims=True)
    ex = jnp.exp(qd - m)
    b_ref[...] = ex / jnp.sum(ex, axis=1, keepdims=True)


def _attn_weights(nodes, edges2d, inv2d, wq, wk):
    grid = N // BN
    return pl.pallas_call(
        _attn_body,
        grid=(grid,),
        in_specs=[
            pl.BlockSpec((BN, D_NODE), lambda i: (i, 0)),
            pl.BlockSpec((BN, K * D_EDGE), lambda i: (i, 0)),
            pl.BlockSpec((BN, 1), lambda i: (i, 0)),
            pl.BlockSpec((D_NODE, D_EDGE), lambda i: (0, 0)),
            pl.BlockSpec((D_EDGE, D_EDGE), lambda i: (0, 0)),
        ],
        out_specs=pl.BlockSpec((BN, K), lambda i: (i, 0)),
        out_shape=jax.ShapeDtypeStruct((N, K), jnp.float32),
    )(nodes, edges2d, inv2d, wq, wk)


def _sc_mix_body(nodes_hbm, nlist_hbm, b_hbm, out_hbm,
                 idx_v, b_v, buf_a, buf_b, out_v, sem_a, sem_b):
    wid = lax.axis_index("s") * NC + lax.axis_index("c")
    base = wid * PER_W

    pltpu.sync_copy(nlist_hbm.at[wid], idx_v)
    pltpu.sync_copy(b_hbm.at[wid], b_v)

    pltpu.async_copy(nodes_hbm.at[idx_v.at[0]], buf_a, sem_a)
    pltpu.async_copy(nodes_hbm.at[idx_v.at[1]], buf_b, sem_b)

    dnums = lax.GatherDimensionNumbers(
        offset_dims=(), collapsed_slice_dims=(0,), start_index_map=(0,))

    def compute(grp, buf):
        def node_body(i, carry):
            node = grp * G + i
            bvec = b_v[pl.ds(node * K, K)]
            bjs = [
                lax.gather(
                    bvec, jnp.full((L, 1), j, jnp.int32), dnums,
                    slice_sizes=(1,),
                    mode=lax.GatherScatterMode.PROMISE_IN_BOUNDS)
                for j in range(K)
            ]
            for d in range(D_NODE // L):
                acc = bjs[0] * buf[i * K, pl.ds(d * L, L)]
                for j in range(1, K):
                    acc = acc + bjs[j] * buf[i * K + j, pl.ds(d * L, L)]
                out_v[node, pl.ds(d * L, L)] = acc
            return carry
        lax.fori_loop(0, G, node_body, 0)

    def iter_body(g, carry):
        for parity, (buf, sem) in enumerate(((buf_a, sem_a), (buf_b, sem_b))):
            grp = 2 * g + parity
            pltpu.make_async_copy(nodes_hbm.at[idx_v.at[grp]], buf, sem).wait()
            compute(grp, buf)
            nxt = grp + 2

            @pl.when(nxt < NGRP)
            def _():
                pltpu.async_copy(nodes_hbm.at[idx_v.at[nxt]], buf, sem)
        return carry

    lax.fori_loop(0, NGRP // 2, iter_body, 0)
    pltpu.sync_copy(out_v, out_hbm.at[pl.ds(base, PER_W)])


def _sc_mix(nodes, nlist_w, b_w):
    mesh = plsc.VectorSubcoreMesh(core_axis_name="c", subcore_axis_name="s")
    kern = functools.partial(
        pl.kernel,
        mesh=mesh,
        out_type=jax.ShapeDtypeStruct((N_PAD, D_NODE), jnp.float32),
        scratch_types=[
            pltpu.VMEM((NGRP, ROWS), jnp.int32),
            pltpu.VMEM((PER_W * K,), jnp.float32),
            pltpu.VMEM((ROWS, D_NODE), jnp.float32),
            pltpu.VMEM((ROWS, D_NODE), jnp.float32),
            pltpu.VMEM((PER_W, D_NODE), jnp.float32),
            pltpu.SemaphoreType.DMA,
            pltpu.SemaphoreType.DMA,
        ],
    )(_sc_mix_body)
    return kern(nodes, nlist_w, b_w)


def _out_body(mixed_ref, wv_ref, out_ref):
    out_ref[...] = jnp.maximum(
        jnp.dot(mixed_ref[...], wv_ref[...], preferred_element_type=jnp.float32),
        0.0)


def _out_proj(mixed, wv):
    grid = N // BN
    return pl.pallas_call(
        _out_body,
        grid=(grid,),
        in_specs=[
            pl.BlockSpec((BN, D_NODE), lambda i: (i, 0)),
            pl.BlockSpec((D_NODE, D_NODE), lambda i: (0, 0)),
        ],
        out_specs=pl.BlockSpec((BN, D_NODE), lambda i: (i, 0)),
        out_shape=jax.ShapeDtypeStruct((N, D_NODE), jnp.float32),
    )(mixed, wv)


def kernel(nodes, nlist, edges, inv_degree, wq, wk, wv):
    edges2d = edges.reshape(N, K * D_EDGE)
    inv2d = inv_degree.reshape(N, 1)

    b = _attn_weights(nodes, edges2d, inv2d, wq, wk)

    nlist32 = nlist.astype(jnp.int32)
    nlist_pad = jnp.pad(nlist32, ((0, N_PAD - N), (0, 0)))
    nlist_w = nlist_pad.reshape(NW, NGRP, ROWS)
    b_pad = jnp.pad(b, ((0, N_PAD - N), (0, 0)))
    b_w = b_pad.reshape(NW, PER_W * K)

    mixed = _sc_mix(nodes, nlist_w, b_w)
    return _out_proj(mixed[:N], wv)


# speedup vs baseline: 1.1404x; 1.0429x over previous
import functools

import jax
import jax.numpy as jnp
from jax import lax
from jax.experimental import pallas as pl
from jax.experimental.pallas import tpu as pltpu
from jax.experimental.pallas import tpu_sc as plsc

N = 10000
K = 16
D_NODE = 256
D_EDGE = 16

L = 16
NC = 2
NS = 16
NW = NC * NS
G = 4
ROWS = G * K
N_PAD = 10240
PER_W = N_PAD // NW
NGRP = PER_W // G
BN = 1000


def _attn_body(nodes_ref, edges_ref, inv_ref, wq_ref, wk_ref, b_ref):
    f32 = jnp.float32
    q = jnp.dot(nodes_ref[...], wq_ref[...], preferred_element_type=f32)
    r = lax.dot_general(q, wk_ref[...], (((1,), (1,)), ((), ())),
                        preferred_element_type=f32)
    r = r * inv_ref[...]
    cmod = lax.broadcasted_iota(jnp.int32, (D_EDGE, K * D_EDGE), 1) % D_EDGE
    drow = lax.broadcasted_iota(jnp.int32, (D_EDGE, K * D_EDGE), 0)
    T = (cmod == drow).astype(f32)
    rt = jnp.dot(r, T, preferred_element_type=f32)
    p = rt * edges_ref[...]
    cdiv = lax.broadcasted_iota(jnp.int32, (K * D_EDGE, K), 0) // D_EDGE
    jcol = lax.broadcasted_iota(jnp.int32, (K * D_EDGE, K), 1)
    S = (cdiv == jcol).astype(f32)
    qd = jnp.dot(p, S, preferred_element_type=f32)
    m = jnp.max(qd, axis=1, keepdims=True)
    ex = jnp.exp(qd - m)
    b_ref[...] = ex / jnp.sum(ex, axis=1, keepdims=True)


def _attn_weights(nodes, edges2d, inv2d, wq, wk):
    grid = N // BN
    return pl.pallas_call(
        _attn_body,
        grid=(grid,),
        in_specs=[
            pl.BlockSpec((BN, D_NODE), lambda i: (i, 0)),
            pl.BlockSpec((BN, K * D_EDGE), lambda i: (i, 0)),
            pl.BlockSpec((BN, 1), lambda i: (i, 0)),
            pl.BlockSpec((D_NODE, D_EDGE), lambda i: (0, 0)),
            pl.BlockSpec((D_EDGE, D_EDGE), lambda i: (0, 0)),
        ],
        out_specs=pl.BlockSpec((BN, K), lambda i: (i, 0)),
        out_shape=jax.ShapeDtypeStruct((N, K), jnp.float32),
    )(nodes, edges2d, inv2d, wq, wk)


def _sc_mix_body(nodes_hbm, nlist_hbm, b_hbm, out_hbm,
                 idx_v, b_v, buf_a, buf_b, out_v, sem_a, sem_b):
    wid = lax.axis_index("s") * NC + lax.axis_index("c")
    base = wid * PER_W

    pltpu.sync_copy(nlist_hbm.at[wid], idx_v)
    pltpu.sync_copy(b_hbm.at[wid], b_v)

    pltpu.async_copy(nodes_hbm.at[idx_v.at[0]], buf_a, sem_a)
    pltpu.async_copy(nodes_hbm.at[idx_v.at[1]], buf_b, sem_b)

    dnums = lax.GatherDimensionNumbers(
        offset_dims=(), collapsed_slice_dims=(0,), start_index_map=(0,))

    def compute(grp, buf):
        for i in range(G):
            node = grp * G + i
            bvec = b_v[pl.ds(node * K, K)]
            bjs = [
                lax.gather(
                    bvec, jnp.full((L, 1), j, jnp.int32), dnums,
                    slice_sizes=(1,),
                    mode=lax.GatherScatterMode.PROMISE_IN_BOUNDS)
                for j in range(K)
            ]
            for d in range(D_NODE // L):
                acc = bjs[0] * buf[i * K, pl.ds(d * L, L)]
                for j in range(1, K):
                    acc = acc + bjs[j] * buf[i * K + j, pl.ds(d * L, L)]
                out_v[node, pl.ds(d * L, L)] = acc

    def iter_body(g, carry):
        for parity, (buf, sem) in enumerate(((buf_a, sem_a), (buf_b, sem_b))):
            grp = 2 * g + parity
            pltpu.make_async_copy(nodes_hbm.at[idx_v.at[grp]], buf, sem).wait()
            compute(grp, buf)
            nxt = grp + 2

            @pl.when(nxt < NGRP)
            def _():
                pltpu.async_copy(nodes_hbm.at[idx_v.at[nxt]], buf, sem)
        return carry

    lax.fori_loop(0, NGRP // 2, iter_body, 0)
    pltpu.sync_copy(out_v, out_hbm.at[pl.ds(base, PER_W)])


def _sc_mix(nodes, nlist_w, b_w):
    mesh = plsc.VectorSubcoreMesh(core_axis_name="c", subcore_axis_name="s")
    kern = functools.partial(
        pl.kernel,
        mesh=mesh,
        out_type=jax.ShapeDtypeStruct((N_PAD, D_NODE), jnp.float32),
        scratch_types=[
            pltpu.VMEM((NGRP, ROWS), jnp.int32),
            pltpu.VMEM((PER_W * K,), jnp.float32),
            pltpu.VMEM((ROWS, D_NODE), jnp.float32),
            pltpu.VMEM((ROWS, D_NODE), jnp.float32),
            pltpu.VMEM((PER_W, D_NODE), jnp.float32),
            pltpu.SemaphoreType.DMA,
            pltpu.SemaphoreType.DMA,
        ],
    )(_sc_mix_body)
    return kern(nodes, nlist_w, b_w)


def _out_body(mixed_ref, wv_ref, out_ref):
    out_ref[...] = jnp.maximum(
        jnp.dot(mixed_ref[...], wv_ref[...], preferred_element_type=jnp.float32),
        0.0)


def _out_proj(mixed_pad, wv):
    grid = N // BN
    return pl.pallas_call(
        _out_body,
        grid=(grid,),
        in_specs=[
            pl.BlockSpec((BN, D_NODE), lambda i: (i, 0)),
            pl.BlockSpec((D_NODE, D_NODE), lambda i: (0, 0)),
        ],
        out_specs=pl.BlockSpec((BN, D_NODE), lambda i: (i, 0)),
        out_shape=jax.ShapeDtypeStruct((N, D_NODE), jnp.float32),
    )(mixed_pad, wv)


def kernel(nodes, nlist, edges, inv_degree, wq, wk, wv):
    edges2d = edges.reshape(N, K * D_EDGE)
    inv2d = inv_degree.reshape(N, 1)

    b = _attn_weights(nodes, edges2d, inv2d, wq, wk)

    nlist32 = nlist.astype(jnp.int32)
    nlist_pad = jnp.pad(nlist32, ((0, N_PAD - N), (0, 0)))
    nlist_w = nlist_pad.reshape(NW, NGRP, ROWS)
    b_pad = jnp.pad(b, ((0, N_PAD - N), (0, 0)))
    b_w = b_pad.reshape(NW, PER_W * K)

    mixed = _sc_mix(nodes, nlist_w, b_w)
    return _out_proj(mixed, wv)
